# baseline (speedup 1.0000x reference)
import functools

import jax
import jax.numpy as jnp
from jax import lax
from jax.experimental import pallas as pl
from jax.experimental.pallas import tpu as pltpu
from jax.experimental.pallas import tpu_sc as plsc

B = 16384
C = 1001
NADV = 1000

NC, NS, L = 2, 16, 16
NW = NC * NS
PW = B // NW
NCHUNK = PW // 128


def _make_sc_gather():
    mesh = plsc.VectorSubcoreMesh(core_axis_name="c", subcore_axis_name="s")

    @functools.partial(
        pl.kernel,
        out_type=jax.ShapeDtypeStruct((B,), jnp.float32),
        mesh=mesh,
        scratch_types=[
            pltpu.VMEM((PW,), jnp.int32),
            pltpu.VMEM((NCHUNK, 128), jnp.int32),
            pltpu.VMEM((PW,), jnp.float32),
            pltpu.SemaphoreType.DMA,
        ],
    )
    def sc_gather(xflat_hbm, a_hbm, out_hbm, a_v, idx_v, g_v, sem):
        wid = lax.axis_index("s") * NC + lax.axis_index("c")
        base = wid * PW
        pltpu.sync_copy(a_hbm.at[pl.ds(base, PW)], a_v)
        lane = lax.iota(jnp.int32, 16)
        for j in range(PW // L):
            rows = (base + j * L) + lane
            idx = rows * C + a_v[pl.ds(j * L, L)]
            idx_v[j // 8, pl.ds((j % 8) * L, L)] = idx
        cps = [
            pltpu.async_copy(
                xflat_hbm.at[idx_v.at[k]], g_v.at[pl.ds(k * 128, 128)], sem
            )
            for k in range(NCHUNK)
        ]
        for cp in cps:
            cp.wait()
        pltpu.sync_copy(g_v, out_hbm.at[pl.ds(base, PW)])

    return sc_gather


_SC_GATHER = _make_sc_gather()

R = 512


def _tc_body(x_ref, g_ref, o_ref):
    xb = x_ref[...]
    total = jnp.sum(xb, axis=1)
    v = xb[:, NADV]
    scale = jnp.float32(1.0 / NADV)
    o_ref[...] = (v * (1.0 + scale) - total * scale + g_ref[:, 0])[:, None]


def kernel(x, a):
    a32 = a.reshape(-1).astype(jnp.int32)
    xflat = x.reshape(-1)
    g = _SC_GATHER(xflat, a32)
    out = pl.pallas_call(
        _tc_body,
        grid=(B // R,),
        in_specs=[
            pl.BlockSpec((R, C), lambda i: (i, 0)),
            pl.BlockSpec((R, 1), lambda i: (i, 0)),
        ],
        out_specs=pl.BlockSpec((R, 1), lambda i: (i, 0)),
        out_shape=jax.ShapeDtypeStruct((B, 1), jnp.float32),
    )(x, g[:, None])
    return out


# speedup vs baseline: 1.7119x; 1.7119x over previous
import jax
import jax.numpy as jnp
from jax import lax
from jax.experimental import pallas as pl

B = 16384
C = 1001
NADV = 1000

R = 512


def _tc_body(x_ref, a_ref, o_ref):
    xb = x_ref[...]
    av = a_ref[...]
    cols = lax.broadcasted_iota(jnp.int32, (R, C), 1)
    s = jnp.float32(1.0 / NADV)
    coef = (cols == av).astype(jnp.float32) + (cols == NADV).astype(jnp.float32) - s
    o_ref[...] = jnp.sum(xb * coef, axis=1)[:, None]


def kernel(x, a):
    a32 = a.astype(jnp.int32)
    out = pl.pallas_call(
        _tc_body,
        grid=(B // R,),
        in_specs=[
            pl.BlockSpec((R, C), lambda i: (i, 0)),
            pl.BlockSpec((R, 1), lambda i: (i, 0)),
        ],
        out_specs=pl.BlockSpec((R, 1), lambda i: (i, 0)),
        out_shape=jax.ShapeDtypeStruct((B, 1), jnp.float32),
    )(x, a32)
    return out
